# Initial kernel scaffold; baseline (speedup 1.0000x reference)
import jax
import jax.numpy as jnp
from jax.experimental import pallas as pl


def kernel(x_list, A, E, W_gcn, b_gcn, conv_w, conv_b):
    raise NotImplementedError("write your pallas kernel here")


# speedup vs baseline: 84.6816x; 84.6816x over previous
import functools

import jax
import jax.numpy as jnp
from jax import lax
from jax.experimental import pallas as pl
from jax.experimental.pallas import tpu as pltpu
from jax.experimental.pallas import tpu_sc as plsc

W = 8
N = 50000
EDGES = 800000
OUT = 32
NC = 2
NS = 16
NW = NC * NS
CHUNK = 128
CPT = 196
EP = NW * CPT * CHUNK
NPAD = 51200
SLICE = NPAD // NS
NBUF = 4
GROUPS = CPT // NBUF
BN = 2000

_mesh = plsc.VectorSubcoreMesh(core_axis_name="c", subcore_axis_name="s")


@functools.partial(
    pl.kernel,
    out_type=jax.ShapeDtypeStruct((NC * NPAD,), jnp.float32),
    mesh=_mesh,
    scratch_types=[
        pltpu.VMEM((CPT, CHUNK), jnp.int32),
        pltpu.VMEM((CPT, CHUNK), jnp.float32),
        pltpu.VMEM_SHARED((NPAD,), jnp.float32),
        pltpu.SemaphoreType.DMA,
    ],
    compiler_params=pltpu.CompilerParams(use_tc_tiling_on_sc=False),
)
def _deg_kernel(cols_hbm, ev_hbm, zeros_hbm, degp_hbm, cols_v, e_v, deg_sh, ssem):
    cid = lax.axis_index("c")
    sid = lax.axis_index("s")
    g = cid * NS + sid
    pltpu.sync_copy(zeros_hbm.at[pl.ds(sid * SLICE, SLICE)],
                    deg_sh.at[pl.ds(sid * SLICE, SLICE)])
    pltpu.sync_copy(cols_hbm.at[g], cols_v)
    pltpu.sync_copy(ev_hbm.at[g], e_v)
    plsc.subcore_barrier()

    def body(j, carry):
        pltpu.async_copy(e_v.at[j], deg_sh.at[cols_v.at[j]], ssem, add=True)
        return carry

    lax.fori_loop(0, CPT, body, 0)
    pltpu.make_async_copy(ev_hbm.at[g], e_v, ssem).wait()
    plsc.subcore_barrier()
    pltpu.sync_copy(deg_sh.at[pl.ds(sid * SLICE, SLICE)],
                    degp_hbm.at[pl.ds(cid * NPAD + sid * SLICE, SLICE)])


@functools.partial(
    pl.kernel,
    out_type=jax.ShapeDtypeStruct((NC, NPAD, 16), jnp.float32),
    mesh=_mesh,
    scratch_types=[
        pltpu.VMEM((CPT, CHUNK), jnp.int32),
        pltpu.VMEM((CPT, CHUNK), jnp.int32),
        pltpu.VMEM((NBUF, CHUNK), jnp.float32),
        pltpu.VMEM((NBUF, CHUNK, 16), jnp.float32),
        pltpu.VMEM((NBUF, CHUNK, 16), jnp.float32),
        pltpu.VMEM_SHARED((NPAD, 16), jnp.float32),
        pltpu.SemaphoreType.DMA((NBUF,)),
        pltpu.SemaphoreType.DMA((NBUF,)),
        pltpu.SemaphoreType.DMA((NBUF,)),
    ],
    compiler_params=pltpu.CompilerParams(use_tc_tiling_on_sc=False),
)
def _msg_kernel(rows_hbm, cols_hbm, ev_hbm, t_hbm, zeros2_hbm, sp_hbm,
                rows_v, cols_v, e_v, tb, ub, s_sh, gsem, ssem, isem):
    cid = lax.axis_index("c")
    sid = lax.axis_index("s")
    g = cid * NS + sid
    pltpu.sync_copy(zeros2_hbm.at[pl.ds(sid * SLICE, SLICE)],
                    s_sh.at[pl.ds(sid * SLICE, SLICE)])
    pltpu.sync_copy(rows_hbm.at[g], rows_v)
    pltpu.sync_copy(cols_hbm.at[g], cols_v)
    plsc.subcore_barrier()

    for b in range(NBUF):
        pltpu.async_copy(t_hbm.at[rows_v.at[b]], tb.at[b], gsem.at[b])
        pltpu.async_copy(ev_hbm.at[g, b], e_v.at[b], isem.at[b])

    def group(it, carry):
        base = it * NBUF
        for b in range(NBUF):
            j = base + b
            pltpu.make_async_copy(zeros2_hbm.at[pl.ds(0, CHUNK)],
                                  tb.at[b], gsem.at[b]).wait()
            pltpu.make_async_copy(ev_hbm.at[g, b], e_v.at[b],
                                  isem.at[b]).wait()

            @pl.when(it > 0)
            def _():
                pltpu.make_async_copy(zeros2_hbm.at[pl.ds(0, CHUNK)],
                                      ub.at[b], ssem.at[b]).wait()

            def rbody(q, carry2):
                base_r = q * 16
                e16 = e_v[b, pl.ds(base_r, 16)]
                for i in range(16):
                    r = base_r + i
                    ub[b, r, :] = e16[i] * tb[b, r, :]
                return carry2

            lax.fori_loop(0, CHUNK // 16, rbody, 0)

            @pl.when(j + NBUF < CPT)
            def _():
                pltpu.async_copy(t_hbm.at[rows_v.at[j + NBUF]],
                                 tb.at[b], gsem.at[b])
                pltpu.async_copy(ev_hbm.at[g, j + NBUF], e_v.at[b],
                                 isem.at[b])

            pltpu.async_copy(ub.at[b], s_sh.at[cols_v.at[j]],
                             ssem.at[b], add=True)
        return carry

    lax.fori_loop(0, GROUPS, group, 0)
    for b in range(NBUF):
        pltpu.make_async_copy(zeros2_hbm.at[pl.ds(0, CHUNK)],
                              ub.at[b], ssem.at[b]).wait()
    plsc.subcore_barrier()
    pltpu.sync_copy(s_sh.at[pl.ds(sid * SLICE, SLICE)],
                    sp_hbm.at[cid, pl.ds(sid * SLICE, SLICE)])


def _prep_body(degp_ref, xt_ref, t_ref):
    deg = degp_ref[0] + degp_ref[1]
    dis = jnp.where(deg > 0, lax.rsqrt(deg), 0.0)
    tx = dis * xt_ref[...]
    t_ref[...] = jnp.concatenate(
        [tx, dis, jnp.zeros((BN, 7), jnp.float32)], axis=1)


def _out_body(sp_ref, t_ref, c2_ref, cb_ref, o_ref):
    dis = t_ref[:, 8:9]
    s = (sp_ref[0, :, 0:8] + sp_ref[1, :, 0:8]) * dis
    z = jnp.dot(s, c2_ref[...], preferred_element_type=jnp.float32)
    z = z + cb_ref[...]
    z = jnp.where(z >= 0, z, 0.01 * z)
    for w in range(W):
        o_ref[w] = z[:, OUT * w:OUT * (w + 1)]


def kernel(x_list, A, E, W_gcn, b_gcn, conv_w, conv_b):
    rows = A[0].astype(jnp.int32)
    cols = A[1].astype(jnp.int32)
    ev = E.astype(jnp.float32)
    pad = EP - EDGES
    rows3 = jnp.concatenate([rows, jnp.zeros((pad,), jnp.int32)]).reshape(NW, CPT, CHUNK)
    cols3 = jnp.concatenate([cols, jnp.zeros((pad,), jnp.int32)]).reshape(NW, CPT, CHUNK)
    ev3 = jnp.concatenate([ev, jnp.zeros((pad,), jnp.float32)]).reshape(NW, CPT, CHUNK)
    zeros1 = jnp.zeros((NPAD,), jnp.float32)
    zeros2 = jnp.zeros((NPAD, 16), jnp.float32)
    xt = x_list[:, :, 0].T

    degp = _deg_kernel(cols3, ev3, zeros1)
    degp3 = degp.reshape(NC, NPAD)[:, :N, None]

    t = pl.pallas_call(
        _prep_body,
        grid=(N // BN,),
        in_specs=[pl.BlockSpec((2, BN, 1), lambda i: (0, i, 0)),
                  pl.BlockSpec((BN, W), lambda i: (i, 0))],
        out_specs=pl.BlockSpec((BN, 16), lambda i: (i, 0)),
        out_shape=jax.ShapeDtypeStruct((N, 16), jnp.float32),
    )(degp3, xt)

    sp = _msg_kernel(rows3, cols3, ev3, t, zeros2)

    wg = W_gcn[:, 0, :]
    pmat = jnp.einsum('ock,jc->jko', conv_w, wg)
    cmat = jnp.zeros((W, W, OUT), jnp.float32)
    for k in range(3):
        for j in range(W):
            w_ = j - k + 1
            if 0 <= w_ < W:
                cmat = cmat.at[j, w_].add(pmat[j, k])
    cb = jnp.tile(conv_b[None, :], (W, 1))
    for k in range(3):
        for w_ in range(W):
            jj = w_ + k - 1
            if 0 <= jj < W:
                cb = cb.at[w_].add(conv_w[:, :, k] @ b_gcn[jj])
    c2 = cmat.reshape(W, W * OUT)
    cb2 = cb.reshape(1, W * OUT)

    out = pl.pallas_call(
        _out_body,
        grid=(N // BN,),
        in_specs=[pl.BlockSpec((2, BN, 16), lambda i: (0, i, 0)),
                  pl.BlockSpec((BN, 16), lambda i: (i, 0)),
                  pl.BlockSpec((W, W * OUT), lambda i: (0, 0)),
                  pl.BlockSpec((1, W * OUT), lambda i: (0, 0))],
        out_specs=pl.BlockSpec((W, BN, OUT), lambda i: (0, i, 0)),
        out_shape=jax.ShapeDtypeStruct((W, N, OUT), jnp.float32),
    )(sp, t, c2, cb2)
    return out


# speedup vs baseline: 113.3761x; 1.3389x over previous
import functools

import jax
import jax.numpy as jnp
from jax import lax
from jax.experimental import pallas as pl
from jax.experimental.pallas import tpu as pltpu
from jax.experimental.pallas import tpu_sc as plsc

W = 8
N = 50000
EDGES = 800000
OUT = 32
NC = 2
NS = 16
NW = NC * NS
CHUNK = 128
CPT = 196
EP = NW * CPT * CHUNK
NPAD = 51200
SLICE = NPAD // NS
NBUF = 4
GROUPS = CPT // NBUF
BN = 2000

_mesh = plsc.VectorSubcoreMesh(core_axis_name="c", subcore_axis_name="s")


@functools.partial(
    pl.kernel,
    out_type=jax.ShapeDtypeStruct((NC * NPAD,), jnp.float32),
    mesh=_mesh,
    scratch_types=[
        pltpu.VMEM((CPT, CHUNK), jnp.int32),
        pltpu.VMEM((CPT, CHUNK), jnp.float32),
        pltpu.VMEM_SHARED((NPAD,), jnp.float32),
        pltpu.SemaphoreType.DMA,
    ],
    compiler_params=pltpu.CompilerParams(use_tc_tiling_on_sc=False),
)
def _deg_kernel(cols_hbm, ev_hbm, zeros_hbm, degp_hbm, cols_v, e_v, deg_sh, ssem):
    cid = lax.axis_index("c")
    sid = lax.axis_index("s")
    g = cid * NS + sid
    pltpu.sync_copy(zeros_hbm.at[pl.ds(sid * SLICE, SLICE)],
                    deg_sh.at[pl.ds(sid * SLICE, SLICE)])
    pltpu.sync_copy(cols_hbm.at[g], cols_v)
    pltpu.sync_copy(ev_hbm.at[g], e_v)
    plsc.subcore_barrier()

    def body(j, carry):
        pltpu.async_copy(e_v.at[j], deg_sh.at[cols_v.at[j]], ssem, add=True)
        return carry

    lax.fori_loop(0, CPT, body, 0)
    pltpu.make_async_copy(ev_hbm.at[g], e_v, ssem).wait()
    plsc.subcore_barrier()
    pltpu.sync_copy(deg_sh.at[pl.ds(sid * SLICE, SLICE)],
                    degp_hbm.at[pl.ds(cid * NPAD + sid * SLICE, SLICE)])


@functools.partial(
    pl.kernel,
    out_type=jax.ShapeDtypeStruct((NC, NPAD, 16), jnp.float32),
    mesh=_mesh,
    scratch_types=[
        pltpu.VMEM((CPT, CHUNK), jnp.int32),
        pltpu.VMEM((CPT, CHUNK), jnp.int32),
        pltpu.VMEM((NBUF, CHUNK), jnp.float32),
        pltpu.VMEM((NBUF, CHUNK, 16), jnp.float32),
        pltpu.VMEM((NBUF, CHUNK, 16), jnp.float32),
        pltpu.VMEM_SHARED((NPAD, 16), jnp.float32),
        pltpu.SemaphoreType.DMA((NBUF,)),
        pltpu.SemaphoreType.DMA((NBUF,)),
        pltpu.SemaphoreType.DMA((NBUF,)),
    ],
    compiler_params=pltpu.CompilerParams(use_tc_tiling_on_sc=False),
)
def _msg_kernel(rows_hbm, cols_hbm, ev_hbm, t_hbm, zeros2_hbm, sp_hbm,
                rows_v, cols_v, e_v, tb, ub, s_sh, gsem, ssem, isem):
    cid = lax.axis_index("c")
    sid = lax.axis_index("s")
    g = cid * NS + sid
    pltpu.sync_copy(zeros2_hbm.at[pl.ds(sid * SLICE, SLICE)],
                    s_sh.at[pl.ds(sid * SLICE, SLICE)])
    pltpu.sync_copy(rows_hbm.at[g], rows_v)
    pltpu.sync_copy(cols_hbm.at[g], cols_v)
    plsc.subcore_barrier()

    for b in range(NBUF):
        pltpu.async_copy(t_hbm.at[rows_v.at[b]], tb.at[b], gsem.at[b])
        pltpu.async_copy(ev_hbm.at[g, b], e_v.at[b], isem.at[b])

    def group(it, carry):
        base = it * NBUF
        for b in range(NBUF):
            j = base + b
            pltpu.make_async_copy(zeros2_hbm.at[pl.ds(0, CHUNK)],
                                  tb.at[b], gsem.at[b]).wait()
            pltpu.make_async_copy(ev_hbm.at[g, b], e_v.at[b],
                                  isem.at[b]).wait()

            @pl.when(it > 0)
            def _():
                pltpu.make_async_copy(zeros2_hbm.at[pl.ds(0, CHUNK)],
                                      ub.at[b], ssem.at[b]).wait()

            def rbody(q, carry2):
                base_r = q * 16
                e16 = e_v[b, pl.ds(base_r, 16)]
                for i in range(16):
                    r = base_r + i
                    ub[b, r, :] = e16[i] * tb[b, r, :]
                return carry2

            lax.fori_loop(0, CHUNK // 16, rbody, 0)

            @pl.when(j + NBUF < CPT)
            def _():
                pltpu.async_copy(t_hbm.at[rows_v.at[j + NBUF]],
                                 tb.at[b], gsem.at[b])
                pltpu.async_copy(ev_hbm.at[g, j + NBUF], e_v.at[b],
                                 isem.at[b])

            pltpu.async_copy(ub.at[b], s_sh.at[cols_v.at[j]],
                             ssem.at[b], add=True)
        return carry

    lax.fori_loop(0, GROUPS, group, 0)
    for b in range(NBUF):
        pltpu.make_async_copy(zeros2_hbm.at[pl.ds(0, CHUNK)],
                              ub.at[b], ssem.at[b]).wait()
    plsc.subcore_barrier()
    pltpu.sync_copy(s_sh.at[pl.ds(sid * SLICE, SLICE)],
                    sp_hbm.at[cid, pl.ds(sid * SLICE, SLICE)])


def _prep_body(degp_ref, xt_ref, t_ref):
    deg = degp_ref[0] + degp_ref[1]
    dis = jnp.where(deg > 0, lax.rsqrt(deg), 0.0)
    tx = dis * xt_ref[...]
    t_ref[...] = jnp.concatenate(
        [tx, dis, jnp.zeros((BN, 7), jnp.float32)], axis=1)


def _out_body(sp_ref, t_ref, c2_ref, cb_ref, o_ref):
    dis = t_ref[:, 8:9]
    s = (sp_ref[0, :, 0:8] + sp_ref[1, :, 0:8]) * dis
    z = jnp.dot(s, c2_ref[...], preferred_element_type=jnp.float32)
    z = z + cb_ref[...]
    z = jnp.where(z >= 0, z, 0.01 * z)
    o_ref[...] = z


def kernel(x_list, A, E, W_gcn, b_gcn, conv_w, conv_b):
    rows = A[0].astype(jnp.int32)
    cols = A[1].astype(jnp.int32)
    ev = E.astype(jnp.float32)
    pad = EP - EDGES
    rows3 = jnp.concatenate([rows, jnp.zeros((pad,), jnp.int32)]).reshape(NW, CPT, CHUNK)
    cols3 = jnp.concatenate([cols, jnp.zeros((pad,), jnp.int32)]).reshape(NW, CPT, CHUNK)
    ev3 = jnp.concatenate([ev, jnp.zeros((pad,), jnp.float32)]).reshape(NW, CPT, CHUNK)
    zeros1 = jnp.zeros((NPAD,), jnp.float32)
    zeros2 = jnp.zeros((NPAD, 16), jnp.float32)
    xt = x_list[:, :, 0].T

    degp = _deg_kernel(cols3, ev3, zeros1)
    degp3 = degp.reshape(NC, NPAD)[:, :N, None]

    t = pl.pallas_call(
        _prep_body,
        grid=(N // BN,),
        in_specs=[pl.BlockSpec((2, BN, 1), lambda i: (0, i, 0)),
                  pl.BlockSpec((BN, W), lambda i: (i, 0))],
        out_specs=pl.BlockSpec((BN, 16), lambda i: (i, 0)),
        out_shape=jax.ShapeDtypeStruct((N, 16), jnp.float32),
    )(degp3, xt)

    sp = _msg_kernel(rows3, cols3, ev3, t, zeros2)

    wg = W_gcn[:, 0, :]
    pmat = jnp.einsum('ock,jc->jko', conv_w, wg)
    cmat = jnp.zeros((W, W, OUT), jnp.float32)
    for k in range(3):
        for j in range(W):
            w_ = j - k + 1
            if 0 <= w_ < W:
                cmat = cmat.at[j, w_].add(pmat[j, k])
    cb = jnp.tile(conv_b[None, :], (W, 1))
    for k in range(3):
        for w_ in range(W):
            jj = w_ + k - 1
            if 0 <= jj < W:
                cb = cb.at[w_].add(conv_w[:, :, k] @ b_gcn[jj])
    c2 = cmat.reshape(W, W * OUT)
    cb2 = cb.reshape(1, W * OUT)

    out = pl.pallas_call(
        _out_body,
        grid=(N // BN,),
        in_specs=[pl.BlockSpec((2, BN, 16), lambda i: (0, i, 0)),
                  pl.BlockSpec((BN, 16), lambda i: (i, 0)),
                  pl.BlockSpec((W, W * OUT), lambda i: (0, 0)),
                  pl.BlockSpec((1, W * OUT), lambda i: (0, 0))],
        out_specs=pl.BlockSpec((BN, W * OUT), lambda i: (i, 0)),
        out_shape=jax.ShapeDtypeStruct((N, W * OUT), jnp.float32),
    )(sp, t, c2, cb2)
    return jnp.transpose(out.reshape(N, W, OUT), (1, 0, 2))


# speedup vs baseline: 116.3335x; 1.0261x over previous
import functools

import jax
import jax.numpy as jnp
from jax import lax
from jax.experimental import pallas as pl
from jax.experimental.pallas import tpu as pltpu
from jax.experimental.pallas import tpu_sc as plsc

W = 8
N = 50000
EDGES = 800000
OUT = 32
NC = 2
NS = 16
NW = NC * NS
CHUNK = 128
CPT = 196
EP = NW * CPT * CHUNK
NPAD = 51200
SLICE = NPAD // NS
NBUF = 4
GROUPS = CPT // NBUF
BN = 2000
BND = 2048

_mesh = plsc.VectorSubcoreMesh(core_axis_name="c", subcore_axis_name="s")


@functools.partial(
    pl.kernel,
    out_type=jax.ShapeDtypeStruct((NC * NPAD,), jnp.float32),
    mesh=_mesh,
    scratch_types=[
        pltpu.VMEM((CPT, CHUNK), jnp.int32),
        pltpu.VMEM((CPT, CHUNK), jnp.float32),
        pltpu.VMEM_SHARED((NPAD,), jnp.float32),
        pltpu.SemaphoreType.DMA,
    ],
    compiler_params=pltpu.CompilerParams(use_tc_tiling_on_sc=False),
)
def _deg_kernel(cols_hbm, ev_hbm, zeros_hbm, degp_hbm, cols_v, e_v, deg_sh, ssem):
    cid = lax.axis_index("c")
    sid = lax.axis_index("s")
    g = cid * NS + sid
    pltpu.sync_copy(zeros_hbm.at[pl.ds(sid * SLICE, SLICE)],
                    deg_sh.at[pl.ds(sid * SLICE, SLICE)])
    pltpu.sync_copy(cols_hbm.at[g], cols_v)
    pltpu.sync_copy(ev_hbm.at[g], e_v)
    plsc.subcore_barrier()

    def body(j, carry):
        pltpu.async_copy(e_v.at[j], deg_sh.at[cols_v.at[j]], ssem, add=True)
        return carry

    lax.fori_loop(0, CPT, body, 0)
    pltpu.make_async_copy(ev_hbm.at[g], e_v, ssem).wait()
    plsc.subcore_barrier()
    pltpu.sync_copy(deg_sh.at[pl.ds(sid * SLICE, SLICE)],
                    degp_hbm.at[pl.ds(cid * NPAD + sid * SLICE, SLICE)])


@functools.partial(
    pl.kernel,
    out_type=jax.ShapeDtypeStruct((NC, NPAD, 16), jnp.float32),
    mesh=_mesh,
    scratch_types=[
        pltpu.VMEM((CPT, CHUNK), jnp.int32),
        pltpu.VMEM((CPT, CHUNK), jnp.int32),
        pltpu.VMEM((NBUF, CHUNK), jnp.float32),
        pltpu.VMEM((NBUF, CHUNK, 16), jnp.float32),
        pltpu.VMEM((NBUF, CHUNK, 16), jnp.float32),
        pltpu.VMEM_SHARED((NPAD, 16), jnp.float32),
        pltpu.SemaphoreType.DMA((NBUF,)),
        pltpu.SemaphoreType.DMA((NBUF,)),
        pltpu.SemaphoreType.DMA((NBUF,)),
    ],
    compiler_params=pltpu.CompilerParams(use_tc_tiling_on_sc=False),
)
def _msg_kernel(rows_hbm, cols_hbm, ev_hbm, t_hbm, zeros2_hbm, sp_hbm,
                rows_v, cols_v, e_v, tb, ub, s_sh, gsem, ssem, isem):
    cid = lax.axis_index("c")
    sid = lax.axis_index("s")
    g = cid * NS + sid
    pltpu.sync_copy(zeros2_hbm.at[pl.ds(sid * SLICE, SLICE)],
                    s_sh.at[pl.ds(sid * SLICE, SLICE)])
    pltpu.sync_copy(rows_hbm.at[g], rows_v)
    pltpu.sync_copy(cols_hbm.at[g], cols_v)
    plsc.subcore_barrier()

    for b in range(NBUF):
        pltpu.async_copy(t_hbm.at[rows_v.at[b]], tb.at[b], gsem.at[b])
        pltpu.async_copy(ev_hbm.at[g, b], e_v.at[b], isem.at[b])

    def group(it, carry):
        base = it * NBUF
        for b in range(NBUF):
            j = base + b
            pltpu.make_async_copy(zeros2_hbm.at[pl.ds(0, CHUNK)],
                                  tb.at[b], gsem.at[b]).wait()
            pltpu.make_async_copy(ev_hbm.at[g, b], e_v.at[b],
                                  isem.at[b]).wait()

            @pl.when(it > 0)
            def _():
                pltpu.make_async_copy(zeros2_hbm.at[pl.ds(0, CHUNK)],
                                      ub.at[b], ssem.at[b]).wait()

            def rbody(q, carry2):
                base_r = q * 16
                e16 = e_v[b, pl.ds(base_r, 16)]
                for i in range(16):
                    r = base_r + i
                    ub[b, r, :] = e16[i] * tb[b, r, :]
                return carry2

            lax.fori_loop(0, CHUNK // 16, rbody, 0)

            @pl.when(j + NBUF < CPT)
            def _():
                pltpu.async_copy(t_hbm.at[rows_v.at[j + NBUF]],
                                 tb.at[b], gsem.at[b])
                pltpu.async_copy(ev_hbm.at[g, j + NBUF], e_v.at[b],
                                 isem.at[b])

            pltpu.async_copy(ub.at[b], s_sh.at[cols_v.at[j]],
                             ssem.at[b], add=True)
        return carry

    lax.fori_loop(0, GROUPS, group, 0)
    for b in range(NBUF):
        pltpu.make_async_copy(zeros2_hbm.at[pl.ds(0, CHUNK)],
                              ub.at[b], ssem.at[b]).wait()
    plsc.subcore_barrier()
    pltpu.sync_copy(s_sh.at[pl.ds(sid * SLICE, SLICE)],
                    sp_hbm.at[cid, pl.ds(sid * SLICE, SLICE)])


def _prep_body(degp_ref, xt_ref, t_ref):
    deg = degp_ref[0:1, :] + degp_ref[1:2, :]
    dis_row = jnp.where(deg > 0, lax.rsqrt(deg), 0.0)
    dis = jnp.transpose(dis_row)
    tx = dis * xt_ref[...]
    t_ref[...] = jnp.concatenate(
        [tx, dis, jnp.zeros((BND, 7), jnp.float32)], axis=1)


def _out_body(sp_ref, t_ref, c2_ref, cb_ref, o_ref):
    dis = t_ref[:, 8:9]
    s = (sp_ref[0, :, 0:8] + sp_ref[1, :, 0:8]) * dis
    z = jnp.dot(s, c2_ref[...], preferred_element_type=jnp.float32)
    z = z + cb_ref[...]
    z = jnp.where(z >= 0, z, 0.01 * z)
    o_ref[...] = z


def kernel(x_list, A, E, W_gcn, b_gcn, conv_w, conv_b):
    rows = A[0].astype(jnp.int32)
    cols = A[1].astype(jnp.int32)
    ev = E.astype(jnp.float32)
    pad = EP - EDGES
    rows3 = jnp.concatenate([rows, jnp.zeros((pad,), jnp.int32)]).reshape(NW, CPT, CHUNK)
    cols3 = jnp.concatenate([cols, jnp.zeros((pad,), jnp.int32)]).reshape(NW, CPT, CHUNK)
    ev3 = jnp.concatenate([ev, jnp.zeros((pad,), jnp.float32)]).reshape(NW, CPT, CHUNK)
    zeros1 = jnp.zeros((NPAD,), jnp.float32)
    zeros2 = jnp.zeros((NPAD, 16), jnp.float32)
    xt = x_list[:, :, 0].T

    degp = _deg_kernel(cols3, ev3, zeros1)
    degp2 = degp.reshape(NC, NPAD)

    xt_p = jnp.zeros((NPAD, W), jnp.float32).at[:N].set(xt)
    t = pl.pallas_call(
        _prep_body,
        grid=(NPAD // BND,),
        in_specs=[pl.BlockSpec((2, BND), lambda i: (0, i)),
                  pl.BlockSpec((BND, W), lambda i: (i, 0))],
        out_specs=pl.BlockSpec((BND, 16), lambda i: (i, 0)),
        out_shape=jax.ShapeDtypeStruct((NPAD, 16), jnp.float32),
    )(degp2, xt_p)

    sp = _msg_kernel(rows3, cols3, ev3, t, zeros2)

    wg = W_gcn[:, 0, :]
    pmat = jnp.einsum('ock,jc->jko', conv_w, wg)
    cmat = jnp.zeros((W, W, OUT), jnp.float32)
    for k in range(3):
        for j in range(W):
            w_ = j - k + 1
            if 0 <= w_ < W:
                cmat = cmat.at[j, w_].add(pmat[j, k])
    cb = jnp.tile(conv_b[None, :], (W, 1))
    for k in range(3):
        for w_ in range(W):
            jj = w_ + k - 1
            if 0 <= jj < W:
                cb = cb.at[w_].add(conv_w[:, :, k] @ b_gcn[jj])
    c2 = cmat.reshape(W, W * OUT)
    cb2 = cb.reshape(1, W * OUT)

    out = pl.pallas_call(
        _out_body,
        grid=(N // BN,),
        in_specs=[pl.BlockSpec((2, BN, 16), lambda i: (0, i, 0)),
                  pl.BlockSpec((BN, 16), lambda i: (i, 0)),
                  pl.BlockSpec((W, W * OUT), lambda i: (0, 0)),
                  pl.BlockSpec((1, W * OUT), lambda i: (0, 0))],
        out_specs=pl.BlockSpec((BN, W * OUT), lambda i: (i, 0)),
        out_shape=jax.ShapeDtypeStruct((N, W * OUT), jnp.float32),
    )(sp, t, c2, cb2)
    return jnp.transpose(out.reshape(N, W, OUT), (1, 0, 2))


# speedup vs baseline: 119.3103x; 1.0256x over previous
import functools

import jax
import jax.numpy as jnp
from jax import lax
from jax.experimental import pallas as pl
from jax.experimental.pallas import tpu as pltpu
from jax.experimental.pallas import tpu_sc as plsc

W = 8
N = 50000
EDGES = 800000
OUT = 32
NC = 2
NS = 16
NW = NC * NS
CHUNK = 128
CPT = 196
EP = NW * CPT * CHUNK
NPAD = 51200
SLICE = NPAD // NS
NBUF = 4
GROUPS = CPT // NBUF
BN = 2000
BND = 2048

_mesh = plsc.VectorSubcoreMesh(core_axis_name="c", subcore_axis_name="s")


@functools.partial(
    pl.kernel,
    out_type=jax.ShapeDtypeStruct((NC * NPAD,), jnp.float32),
    mesh=_mesh,
    scratch_types=[
        pltpu.VMEM((CPT, CHUNK), jnp.int32),
        pltpu.VMEM((CPT, CHUNK), jnp.float32),
        pltpu.VMEM_SHARED((NPAD,), jnp.float32),
        pltpu.SemaphoreType.DMA,
    ],
    compiler_params=pltpu.CompilerParams(use_tc_tiling_on_sc=False),
)
def _deg_kernel(rc_hbm, ev_hbm, zeros_hbm, degp_hbm, cols_v, e_v, deg_sh, ssem):
    cid = lax.axis_index("c")
    sid = lax.axis_index("s")
    g = cid * NS + sid
    pltpu.sync_copy(zeros_hbm, deg_sh.at[pl.ds(sid * SLICE, SLICE)])
    pltpu.sync_copy(rc_hbm.at[1, g], cols_v)
    pltpu.sync_copy(ev_hbm.at[g], e_v)
    plsc.subcore_barrier()

    def body(j, carry):
        pltpu.async_copy(e_v.at[j], deg_sh.at[cols_v.at[j]], ssem, add=True)
        return carry

    lax.fori_loop(0, CPT, body, 0)
    pltpu.make_async_copy(ev_hbm.at[g], e_v, ssem).wait()
    plsc.subcore_barrier()
    pltpu.sync_copy(deg_sh.at[pl.ds(sid * SLICE, SLICE)],
                    degp_hbm.at[pl.ds(cid * NPAD + sid * SLICE, SLICE)])


@functools.partial(
    pl.kernel,
    out_type=jax.ShapeDtypeStruct((NC, NPAD, 16), jnp.float32),
    mesh=_mesh,
    scratch_types=[
        pltpu.VMEM((CPT, CHUNK), jnp.int32),
        pltpu.VMEM((CPT, CHUNK), jnp.int32),
        pltpu.VMEM((NBUF, CHUNK), jnp.float32),
        pltpu.VMEM((NBUF, CHUNK, 16), jnp.float32),
        pltpu.VMEM((NBUF, CHUNK, 16), jnp.float32),
        pltpu.VMEM_SHARED((NPAD, 16), jnp.float32),
        pltpu.SemaphoreType.DMA((NBUF,)),
        pltpu.SemaphoreType.DMA((NBUF,)),
        pltpu.SemaphoreType.DMA((NBUF,)),
    ],
    compiler_params=pltpu.CompilerParams(use_tc_tiling_on_sc=False),
)
def _msg_kernel(rc_hbm, ev_hbm, t_hbm, zeros2_hbm, sp_hbm,
                rows_v, cols_v, e_v, tb, ub, s_sh, gsem, ssem, isem):
    cid = lax.axis_index("c")
    sid = lax.axis_index("s")
    g = cid * NS + sid
    pltpu.sync_copy(zeros2_hbm, s_sh.at[pl.ds(sid * SLICE, SLICE)])
    pltpu.sync_copy(rc_hbm.at[0, g], rows_v)
    pltpu.sync_copy(rc_hbm.at[1, g], cols_v)
    plsc.subcore_barrier()

    for b in range(NBUF):
        pltpu.async_copy(t_hbm.at[rows_v.at[b]], tb.at[b], gsem.at[b])
        pltpu.async_copy(ev_hbm.at[g, b], e_v.at[b], isem.at[b])

    def group(it, carry):
        base = it * NBUF
        for b in range(NBUF):
            j = base + b
            pltpu.make_async_copy(zeros2_hbm.at[pl.ds(0, CHUNK)],
                                  tb.at[b], gsem.at[b]).wait()
            pltpu.make_async_copy(ev_hbm.at[g, b], e_v.at[b],
                                  isem.at[b]).wait()

            @pl.when(it > 0)
            def _():
                pltpu.make_async_copy(zeros2_hbm.at[pl.ds(0, CHUNK)],
                                      ub.at[b], ssem.at[b]).wait()

            def rbody(q, carry2):
                base_r = q * 16
                e16 = e_v[b, pl.ds(base_r, 16)]
                for i in range(16):
                    r = base_r + i
                    ub[b, r, :] = e16[i] * tb[b, r, :]
                return carry2

            lax.fori_loop(0, CHUNK // 16, rbody, 0)

            @pl.when(j + NBUF < CPT)
            def _():
                pltpu.async_copy(t_hbm.at[rows_v.at[j + NBUF]],
                                 tb.at[b], gsem.at[b])
                pltpu.async_copy(ev_hbm.at[g, j + NBUF], e_v.at[b],
                                 isem.at[b])

            pltpu.async_copy(ub.at[b], s_sh.at[cols_v.at[j]],
                             ssem.at[b], add=True)
        return carry

    lax.fori_loop(0, GROUPS, group, 0)
    for b in range(NBUF):
        pltpu.make_async_copy(zeros2_hbm.at[pl.ds(0, CHUNK)],
                              ub.at[b], ssem.at[b]).wait()
    plsc.subcore_barrier()
    pltpu.sync_copy(s_sh.at[pl.ds(sid * SLICE, SLICE)],
                    sp_hbm.at[cid, pl.ds(sid * SLICE, SLICE)])


def _prep_body(degp_ref, xt_ref, t_ref):
    ones21 = jnp.ones((2, 1), jnp.float32)
    deg = lax.dot_general(degp_ref[...], ones21,
                          dimension_numbers=(((0,), (0,)), ((), ())),
                          preferred_element_type=jnp.float32)
    dis = jnp.where(deg > 0, lax.rsqrt(deg), 0.0)
    g16 = jnp.concatenate(
        [xt_ref[...], jnp.ones((BND, 1), jnp.float32),
         jnp.zeros((BND, 7), jnp.float32)], axis=1)
    t_ref[...] = dis * g16


def _out_body(sp_ref, t_ref, c2_ref, cb_ref, o_ref):
    dis = t_ref[:, 8:9]
    s = (sp_ref[0, :, 0:8] + sp_ref[1, :, 0:8]) * dis
    z = jnp.dot(s, c2_ref[...], preferred_element_type=jnp.float32)
    z = z + cb_ref[...]
    z = jnp.where(z >= 0, z, 0.01 * z)
    o_ref[...] = z


def kernel(x_list, A, E, W_gcn, b_gcn, conv_w, conv_b):
    pad = EP - EDGES
    rc3 = jnp.pad(A.astype(jnp.int32), ((0, 0), (0, pad))).reshape(2, NW, CPT, CHUNK)
    ev3 = jnp.pad(E.astype(jnp.float32), (0, pad)).reshape(NW, CPT, CHUNK)
    zeros1 = jnp.zeros((SLICE,), jnp.float32)
    zeros2 = jnp.zeros((SLICE, 16), jnp.float32)
    rc3, ev3, x_listb = lax.optimization_barrier((rc3, ev3, x_list))
    xt = x_listb[:, :, 0].T

    degp = _deg_kernel(rc3, ev3, zeros1)
    degp2 = degp.reshape(NC, NPAD)

    xt_p = jnp.zeros((NPAD, W), jnp.float32).at[:N].set(xt)
    t = pl.pallas_call(
        _prep_body,
        grid=(NPAD // BND,),
        in_specs=[pl.BlockSpec((2, BND), lambda i: (0, i)),
                  pl.BlockSpec((BND, W), lambda i: (i, 0))],
        out_specs=pl.BlockSpec((BND, 16), lambda i: (i, 0)),
        out_shape=jax.ShapeDtypeStruct((NPAD, 16), jnp.float32),
    )(degp2, xt_p)

    sp = _msg_kernel(rc3, ev3, t, zeros2)

    wg = W_gcn[:, 0, :]
    pmat = jnp.einsum('ock,jc->jko', conv_w, wg)
    cmat = jnp.zeros((W, W, OUT), jnp.float32)
    for k in range(3):
        for j in range(W):
            w_ = j - k + 1
            if 0 <= w_ < W:
                cmat = cmat.at[j, w_].add(pmat[j, k])
    cb = jnp.tile(conv_b[None, :], (W, 1))
    for k in range(3):
        for w_ in range(W):
            jj = w_ + k - 1
            if 0 <= jj < W:
                cb = cb.at[w_].add(conv_w[:, :, k] @ b_gcn[jj])
    c2 = cmat.reshape(W, W * OUT)
    cb2 = cb.reshape(1, W * OUT)

    out = pl.pallas_call(
        _out_body,
        grid=(N // BN,),
        in_specs=[pl.BlockSpec((2, BN, 16), lambda i: (0, i, 0)),
                  pl.BlockSpec((BN, 16), lambda i: (i, 0)),
                  pl.BlockSpec((W, W * OUT), lambda i: (0, 0)),
                  pl.BlockSpec((1, W * OUT), lambda i: (0, 0))],
        out_specs=pl.BlockSpec((BN, W * OUT), lambda i: (i, 0)),
        out_shape=jax.ShapeDtypeStruct((N, W * OUT), jnp.float32),
    )(sp, t, c2, cb2)
    return jnp.transpose(out.reshape(N, W, OUT), (1, 0, 2))


# speedup vs baseline: 119.5027x; 1.0016x over previous
import functools

import jax
import jax.numpy as jnp
from jax import lax
from jax.experimental import pallas as pl
from jax.experimental.pallas import tpu as pltpu
from jax.experimental.pallas import tpu_sc as plsc

W = 8
N = 50000
EDGES = 800000
OUT = 32
NC = 2
NS = 16
NW = NC * NS
CHUNK = 128
CPT = 196
EP = NW * CPT * CHUNK
NPAD = 51200
SLICE = NPAD // NS
NBUF = 4
GROUPS = CPT // NBUF
BN = 2000
BND = 2048

_mesh = plsc.VectorSubcoreMesh(core_axis_name="c", subcore_axis_name="s")


@functools.partial(
    pl.kernel,
    out_type=jax.ShapeDtypeStruct((NC * NPAD,), jnp.float32),
    mesh=_mesh,
    scratch_types=[
        pltpu.VMEM((CPT, CHUNK), jnp.int32),
        pltpu.VMEM((CPT, CHUNK), jnp.float32),
        pltpu.VMEM_SHARED((NPAD,), jnp.float32),
        pltpu.SemaphoreType.DMA,
    ],
    compiler_params=pltpu.CompilerParams(use_tc_tiling_on_sc=False),
)
def _deg_kernel(rc_hbm, ev_hbm, zeros_hbm, degp_hbm, cols_v, e_v, deg_sh, ssem):
    cid = lax.axis_index("c")
    sid = lax.axis_index("s")
    g = cid * NS + sid
    pltpu.sync_copy(zeros_hbm, deg_sh.at[pl.ds(sid * SLICE, SLICE)])
    pltpu.sync_copy(rc_hbm.at[1, g], cols_v)
    pltpu.sync_copy(ev_hbm.at[g], e_v)
    plsc.subcore_barrier()

    def body(j, carry):
        pltpu.async_copy(e_v.at[j], deg_sh.at[cols_v.at[j]], ssem, add=True)
        return carry

    lax.fori_loop(0, CPT, body, 0)
    pltpu.make_async_copy(ev_hbm.at[g], e_v, ssem).wait()
    plsc.subcore_barrier()
    pltpu.sync_copy(deg_sh.at[pl.ds(sid * SLICE, SLICE)],
                    degp_hbm.at[pl.ds(cid * NPAD + sid * SLICE, SLICE)])


@functools.partial(
    pl.kernel,
    out_type=jax.ShapeDtypeStruct((NC, NPAD, 16), jnp.float32),
    mesh=_mesh,
    scratch_types=[
        pltpu.VMEM((CPT, CHUNK), jnp.int32),
        pltpu.VMEM((CPT, CHUNK), jnp.int32),
        pltpu.VMEM((NBUF, CHUNK), jnp.float32),
        pltpu.VMEM((NBUF, CHUNK, 16), jnp.float32),
        pltpu.VMEM((NBUF, CHUNK, 16), jnp.float32),
        pltpu.VMEM_SHARED((NPAD, 16), jnp.float32),
        pltpu.SemaphoreType.DMA((NBUF,)),
        pltpu.SemaphoreType.DMA((NBUF,)),
        pltpu.SemaphoreType.DMA((NBUF,)),
    ],
    compiler_params=pltpu.CompilerParams(use_tc_tiling_on_sc=False),
)
def _msg_kernel(rc_hbm, ev_hbm, t_hbm, zeros2_hbm, sp_hbm,
                rows_v, cols_v, e_v, tb, ub, s_sh, gsem, ssem, isem):
    cid = lax.axis_index("c")
    sid = lax.axis_index("s")
    g = cid * NS + sid
    pltpu.sync_copy(zeros2_hbm, s_sh.at[pl.ds(sid * SLICE, SLICE)])
    pltpu.sync_copy(rc_hbm.at[0, g], rows_v)
    pltpu.sync_copy(rc_hbm.at[1, g], cols_v)
    plsc.subcore_barrier()

    for b in range(NBUF):
        pltpu.async_copy(t_hbm.at[rows_v.at[b]], tb.at[b], gsem.at[b])
        pltpu.async_copy(ev_hbm.at[g, b], e_v.at[b], isem.at[b])

    def group(it, carry):
        base = it * NBUF
        for b in range(NBUF):
            j = base + b
            pltpu.make_async_copy(zeros2_hbm.at[pl.ds(0, CHUNK)],
                                  tb.at[b], gsem.at[b]).wait()
            pltpu.make_async_copy(ev_hbm.at[g, b], e_v.at[b],
                                  isem.at[b]).wait()

            @pl.when(it > 0)
            def _():
                pltpu.make_async_copy(zeros2_hbm.at[pl.ds(0, CHUNK)],
                                      ub.at[b], ssem.at[b]).wait()

            for q in range(CHUNK // 16):
                e16 = e_v[b, pl.ds(q * 16, 16)]
                for i in range(16):
                    r = q * 16 + i
                    ub[b, r, :] = e16[i] * tb[b, r, :]

            @pl.when(j + NBUF < CPT)
            def _():
                pltpu.async_copy(t_hbm.at[rows_v.at[j + NBUF]],
                                 tb.at[b], gsem.at[b])
                pltpu.async_copy(ev_hbm.at[g, j + NBUF], e_v.at[b],
                                 isem.at[b])

            pltpu.async_copy(ub.at[b], s_sh.at[cols_v.at[j]],
                             ssem.at[b], add=True)
        return carry

    lax.fori_loop(0, GROUPS, group, 0)
    for b in range(NBUF):
        pltpu.make_async_copy(zeros2_hbm.at[pl.ds(0, CHUNK)],
                              ub.at[b], ssem.at[b]).wait()
    plsc.subcore_barrier()
    pltpu.sync_copy(s_sh.at[pl.ds(sid * SLICE, SLICE)],
                    sp_hbm.at[cid, pl.ds(sid * SLICE, SLICE)])


def _prep_body(degp_ref, xt_ref, t_ref):
    ones21 = jnp.ones((2, 1), jnp.float32)
    deg = lax.dot_general(degp_ref[...], ones21,
                          dimension_numbers=(((0,), (0,)), ((), ())),
                          preferred_element_type=jnp.float32)
    dis = jnp.where(deg > 0, lax.rsqrt(deg), 0.0)
    g16 = jnp.concatenate(
        [xt_ref[...], jnp.ones((BND, 1), jnp.float32),
         jnp.zeros((BND, 7), jnp.float32)], axis=1)
    t_ref[...] = dis * g16


def _out_body(sp_ref, t_ref, c2_ref, cb_ref, o_ref):
    dis = t_ref[:, 8:9]
    s = (sp_ref[0, :, 0:8] + sp_ref[1, :, 0:8]) * dis
    z = jnp.dot(s, c2_ref[...], preferred_element_type=jnp.float32)
    z = z + cb_ref[...]
    z = jnp.where(z >= 0, z, 0.01 * z)
    o_ref[...] = z


def kernel(x_list, A, E, W_gcn, b_gcn, conv_w, conv_b):
    pad = EP - EDGES
    rc3 = jnp.pad(A.astype(jnp.int32), ((0, 0), (0, pad))).reshape(2, NW, CPT, CHUNK)
    ev3 = jnp.pad(E.astype(jnp.float32), (0, pad)).reshape(NW, CPT, CHUNK)
    zeros1 = jnp.zeros((SLICE,), jnp.float32)
    zeros2 = jnp.zeros((SLICE, 16), jnp.float32)
    rc3, ev3, x_listb = lax.optimization_barrier((rc3, ev3, x_list))
    xt = x_listb[:, :, 0].T

    degp = _deg_kernel(rc3, ev3, zeros1)
    degp2 = degp.reshape(NC, NPAD)

    xt_p = jnp.zeros((NPAD, W), jnp.float32).at[:N].set(xt)
    t = pl.pallas_call(
        _prep_body,
        grid=(NPAD // BND,),
        in_specs=[pl.BlockSpec((2, BND), lambda i: (0, i)),
                  pl.BlockSpec((BND, W), lambda i: (i, 0))],
        out_specs=pl.BlockSpec((BND, 16), lambda i: (i, 0)),
        out_shape=jax.ShapeDtypeStruct((NPAD, 16), jnp.float32),
    )(degp2, xt_p)

    sp = _msg_kernel(rc3, ev3, t, zeros2)

    wg = W_gcn[:, 0, :]
    pmat = jnp.einsum('ock,jc->jko', conv_w, wg)
    cmat = jnp.zeros((W, W, OUT), jnp.float32)
    for k in range(3):
        for j in range(W):
            w_ = j - k + 1
            if 0 <= w_ < W:
                cmat = cmat.at[j, w_].add(pmat[j, k])
    cb = jnp.tile(conv_b[None, :], (W, 1))
    for k in range(3):
        for w_ in range(W):
            jj = w_ + k - 1
            if 0 <= jj < W:
                cb = cb.at[w_].add(conv_w[:, :, k] @ b_gcn[jj])
    c2 = cmat.reshape(W, W * OUT)
    cb2 = cb.reshape(1, W * OUT)

    out = pl.pallas_call(
        _out_body,
        grid=(N // BN,),
        in_specs=[pl.BlockSpec((2, BN, 16), lambda i: (0, i, 0)),
                  pl.BlockSpec((BN, 16), lambda i: (i, 0)),
                  pl.BlockSpec((W, W * OUT), lambda i: (0, 0)),
                  pl.BlockSpec((1, W * OUT), lambda i: (0, 0))],
        out_specs=pl.BlockSpec((BN, W * OUT), lambda i: (i, 0)),
        out_shape=jax.ShapeDtypeStruct((N, W * OUT), jnp.float32),
    )(sp, t, c2, cb2)
    return jnp.transpose(out.reshape(N, W, OUT), (1, 0, 2))


# speedup vs baseline: 120.9086x; 1.0118x over previous
import functools

import jax
import jax.numpy as jnp
from jax import lax
from jax.experimental import pallas as pl
from jax.experimental.pallas import tpu as pltpu
from jax.experimental.pallas import tpu_sc as plsc

W = 8
N = 50000
EDGES = 800000
OUT = 32
NC = 2
NS = 16
NW = NC * NS
CHUNK = 128
CPT = 196
EP = NW * CPT * CHUNK
NPAD = 51200
SLICE = NPAD // NS
NBUF = 7
GROUPS = CPT // NBUF
BN = 2000
BND = 2048

_mesh = plsc.VectorSubcoreMesh(core_axis_name="c", subcore_axis_name="s")


@functools.partial(
    pl.kernel,
    out_type=jax.ShapeDtypeStruct((NC * NPAD,), jnp.float32),
    mesh=_mesh,
    scratch_types=[
        pltpu.VMEM((CPT, CHUNK), jnp.int32),
        pltpu.VMEM((CPT, CHUNK), jnp.float32),
        pltpu.VMEM_SHARED((NPAD,), jnp.float32),
        pltpu.SemaphoreType.DMA,
    ],
    compiler_params=pltpu.CompilerParams(use_tc_tiling_on_sc=False),
)
def _deg_kernel(rc_hbm, ev_hbm, zeros_hbm, degp_hbm, cols_v, e_v, deg_sh, ssem):
    cid = lax.axis_index("c")
    sid = lax.axis_index("s")
    g = cid * NS + sid
    pltpu.sync_copy(zeros_hbm, deg_sh.at[pl.ds(sid * SLICE, SLICE)])
    pltpu.sync_copy(rc_hbm.at[1, g], cols_v)
    pltpu.sync_copy(ev_hbm.at[g], e_v)
    plsc.subcore_barrier()

    def body(j, carry):
        pltpu.async_copy(e_v.at[j], deg_sh.at[cols_v.at[j]], ssem, add=True)
        return carry

    lax.fori_loop(0, CPT, body, 0)
    pltpu.make_async_copy(ev_hbm.at[g], e_v, ssem).wait()
    plsc.subcore_barrier()
    pltpu.sync_copy(deg_sh.at[pl.ds(sid * SLICE, SLICE)],
                    degp_hbm.at[pl.ds(cid * NPAD + sid * SLICE, SLICE)])


@functools.partial(
    pl.kernel,
    out_type=jax.ShapeDtypeStruct((NC, NPAD, 16), jnp.float32),
    mesh=_mesh,
    scratch_types=[
        pltpu.VMEM((CPT, CHUNK), jnp.int32),
        pltpu.VMEM((CPT, CHUNK), jnp.int32),
        pltpu.VMEM((NBUF, CHUNK), jnp.float32),
        pltpu.VMEM((NBUF, CHUNK, 16), jnp.float32),
        pltpu.VMEM((NBUF, CHUNK, 16), jnp.float32),
        pltpu.VMEM_SHARED((NPAD, 16), jnp.float32),
        pltpu.SemaphoreType.DMA((NBUF,)),
        pltpu.SemaphoreType.DMA((NBUF,)),
        pltpu.SemaphoreType.DMA((NBUF,)),
    ],
    compiler_params=pltpu.CompilerParams(use_tc_tiling_on_sc=False),
)
def _msg_kernel(rc_hbm, ev_hbm, t_hbm, zeros2_hbm, sp_hbm,
                rows_v, cols_v, e_v, tb, ub, s_sh, gsem, ssem, isem):
    cid = lax.axis_index("c")
    sid = lax.axis_index("s")
    g = cid * NS + sid
    pltpu.sync_copy(zeros2_hbm, s_sh.at[pl.ds(sid * SLICE, SLICE)])
    pltpu.sync_copy(rc_hbm.at[0, g], rows_v)
    pltpu.sync_copy(rc_hbm.at[1, g], cols_v)
    plsc.subcore_barrier()

    for b in range(NBUF):
        pltpu.async_copy(t_hbm.at[rows_v.at[b]], tb.at[b], gsem.at[b])
        pltpu.async_copy(ev_hbm.at[g, b], e_v.at[b], isem.at[b])

    def group(it, carry):
        base = it * NBUF
        for b in range(NBUF):
            j = base + b
            pltpu.make_async_copy(zeros2_hbm.at[pl.ds(0, CHUNK)],
                                  tb.at[b], gsem.at[b]).wait()
            pltpu.make_async_copy(ev_hbm.at[g, b], e_v.at[b],
                                  isem.at[b]).wait()

            @pl.when(it > 0)
            def _():
                pltpu.make_async_copy(zeros2_hbm.at[pl.ds(0, CHUNK)],
                                      ub.at[b], ssem.at[b]).wait()

            for q in range(CHUNK // 16):
                e16 = e_v[b, pl.ds(q * 16, 16)]
                for i in range(16):
                    r = q * 16 + i
                    ub[b, r, :] = e16[i] * tb[b, r, :]

            @pl.when(j + NBUF < CPT)
            def _():
                pltpu.async_copy(t_hbm.at[rows_v.at[j + NBUF]],
                                 tb.at[b], gsem.at[b])
                pltpu.async_copy(ev_hbm.at[g, j + NBUF], e_v.at[b],
                                 isem.at[b])

            pltpu.async_copy(ub.at[b], s_sh.at[cols_v.at[j]],
                             ssem.at[b], add=True)
        return carry

    lax.fori_loop(0, GROUPS, group, 0)
    for b in range(NBUF):
        pltpu.make_async_copy(zeros2_hbm.at[pl.ds(0, CHUNK)],
                              ub.at[b], ssem.at[b]).wait()
    plsc.subcore_barrier()
    pltpu.sync_copy(s_sh.at[pl.ds(sid * SLICE, SLICE)],
                    sp_hbm.at[cid, pl.ds(sid * SLICE, SLICE)])


def _prep_body(degp_ref, xt_ref, t_ref):
    ones21 = jnp.ones((2, 1), jnp.float32)
    deg = lax.dot_general(degp_ref[...], ones21,
                          dimension_numbers=(((0,), (0,)), ((), ())),
                          preferred_element_type=jnp.float32)
    dis = jnp.where(deg > 0, lax.rsqrt(deg), 0.0)
    g16 = jnp.concatenate(
        [xt_ref[...], jnp.ones((BND, 1), jnp.float32),
         jnp.zeros((BND, 7), jnp.float32)], axis=1)
    t_ref[...] = dis * g16


def _out_body(sp_ref, t_ref, c2_ref, cb_ref, o_ref):
    dis = t_ref[:, 8:9]
    s = (sp_ref[0, :, 0:8] + sp_ref[1, :, 0:8]) * dis
    z = jnp.dot(s, c2_ref[...], preferred_element_type=jnp.float32)
    z = z + cb_ref[...]
    z = jnp.where(z >= 0, z, 0.01 * z)
    o_ref[...] = z


def kernel(x_list, A, E, W_gcn, b_gcn, conv_w, conv_b):
    pad = EP - EDGES
    rc3 = jnp.pad(A.astype(jnp.int32), ((0, 0), (0, pad))).reshape(2, NW, CPT, CHUNK)
    ev3 = jnp.pad(E.astype(jnp.float32), (0, pad)).reshape(NW, CPT, CHUNK)
    zeros1 = jnp.zeros((SLICE,), jnp.float32)
    zeros2 = jnp.zeros((SLICE, 16), jnp.float32)
    rc3, ev3, x_listb = lax.optimization_barrier((rc3, ev3, x_list))
    xt = x_listb[:, :, 0].T

    degp = _deg_kernel(rc3, ev3, zeros1)
    degp2 = degp.reshape(NC, NPAD)

    xt_p = jnp.zeros((NPAD, W), jnp.float32).at[:N].set(xt)
    t = pl.pallas_call(
        _prep_body,
        grid=(NPAD // BND,),
        in_specs=[pl.BlockSpec((2, BND), lambda i: (0, i)),
                  pl.BlockSpec((BND, W), lambda i: (i, 0))],
        out_specs=pl.BlockSpec((BND, 16), lambda i: (i, 0)),
        out_shape=jax.ShapeDtypeStruct((NPAD, 16), jnp.float32),
    )(degp2, xt_p)

    sp = _msg_kernel(rc3, ev3, t, zeros2)

    wg = W_gcn[:, 0, :]
    pmat = jnp.einsum('ock,jc->jko', conv_w, wg)
    cmat = jnp.zeros((W, W, OUT), jnp.float32)
    for k in range(3):
        for j in range(W):
            w_ = j - k + 1
            if 0 <= w_ < W:
                cmat = cmat.at[j, w_].add(pmat[j, k])
    cb = jnp.tile(conv_b[None, :], (W, 1))
    for k in range(3):
        for w_ in range(W):
            jj = w_ + k - 1
            if 0 <= jj < W:
                cb = cb.at[w_].add(conv_w[:, :, k] @ b_gcn[jj])
    c2 = cmat.reshape(W, W * OUT)
    cb2 = cb.reshape(1, W * OUT)

    out = pl.pallas_call(
        _out_body,
        grid=(N // BN,),
        in_specs=[pl.BlockSpec((2, BN, 16), lambda i: (0, i, 0)),
                  pl.BlockSpec((BN, 16), lambda i: (i, 0)),
                  pl.BlockSpec((W, W * OUT), lambda i: (0, 0)),
                  pl.BlockSpec((1, W * OUT), lambda i: (0, 0))],
        out_specs=pl.BlockSpec((BN, W * OUT), lambda i: (i, 0)),
        out_shape=jax.ShapeDtypeStruct((N, W * OUT), jnp.float32),
    )(sp, t, c2, cb2)
    return jnp.transpose(out.reshape(N, W, OUT), (1, 0, 2))
